# baseline (speedup 1.0000x reference)
import functools

import jax
import jax.numpy as jnp
from jax import lax
from jax.experimental import pallas as pl
from jax.experimental.pallas import tpu as pltpu
from jax.experimental.pallas import tpu_sc as plsc

VOCAB = 1000000
MAXLEN = 200
EMBED_DIM = 64
BATCH = 4096

NUM_CORES = 2
NUM_SUBCORES = 16
LANES = 16
NW = NUM_CORES * NUM_SUBCORES
BCH = BATCH // NW
DQ = EMBED_DIM // LANES
PITCH = BCH + 1
PADW = 128
NBUF = 4
TBUF = 2
NGROUPS = MAXLEN // NBUF
LH = MAXLEN // 8


def _make_kernel():
    mesh = plsc.VectorSubcoreMesh(core_axis_name="c", subcore_axis_name="s")

    @functools.partial(
        pl.kernel,
        out_type=jax.ShapeDtypeStruct((MAXLEN, 8, NW * 8, BCH), jnp.float32),
        name="tok_pos_embed",
        mesh=mesh,
        scratch_types=[
            pltpu.VMEM((EMBED_DIM, MAXLEN), jnp.float32),
            pltpu.VMEM((NBUF, BCH), jnp.int32),
            pltpu.VMEM((NBUF, BCH, PADW), jnp.float32),
            pltpu.VMEM((TBUF, 8, 8, PITCH), jnp.float32),
            pltpu.SemaphoreType.DMA,
            pltpu.SemaphoreType.DMA,
            pltpu.SemaphoreType.DMA,
            pltpu.SemaphoreType.DMA,
            pltpu.SemaphoreType.DMA,
            pltpu.SemaphoreType.DMA,
            pltpu.SemaphoreType.DMA,
            pltpu.SemaphoreType.DMA,
            pltpu.SemaphoreType.DMA,
            pltpu.SemaphoreType.DMA,
        ],
        compiler_params=pltpu.CompilerParams(use_tc_tiling_on_sc=True,
                                             needs_layout_passes=False),
    )
    def tok_pos_embed(x_hbm, tok_hbm, pos_hbm, out_hbm,
                      pos_v, idx_v, gbuf, tbuf,
                      g0, g1, g2, g3, o0, o1, i0, i1, i2, i3):
        wid = lax.axis_index("s") * NUM_CORES + lax.axis_index("c")
        gsem = (g0, g1, g2, g3)
        osem = (o0, o1)
        isem = (i0, i1, i2, i3)
        pltpu.sync_copy(pos_hbm, pos_v)

        def idx_src(l):
            return x_hbm.at[l >> 3, wid * 8 + (l & 7), :]

        def start_idx(l, bb):
            pltpu.async_copy(idx_src(l), idx_v.at[bb], isem[bb])

        def wait_idx(l, bb):
            pltpu.make_async_copy(idx_src(l), idx_v.at[bb], isem[bb]).wait()

        def start_gather(bb):
            pltpu.async_copy(tok_hbm.at[idx_v.at[bb]], gbuf.at[bb], gsem[bb])

        def wait_gather(bb):
            pltpu.make_async_copy(tok_hbm.at[idx_v.at[bb]],
                                  gbuf.at[bb], gsem[bb]).wait()

        for bb in range(NBUF):
            start_idx(bb, bb)
        for bb in range(NBUF - 1):
            wait_idx(bb, bb)
            start_gather(bb)

        rows_hi = [(lax.iota(jnp.int32, LANES) + dq * LANES) >> 3
                   for dq in range(DQ)]
        rows_lo = [(lax.iota(jnp.int32, LANES) + dq * LANES) & 7
                   for dq in range(DQ)]
        rows_dq = [lax.iota(jnp.int32, LANES) + dq * LANES for dq in range(DQ)]

        def group_body(g, carry):
            for bb in range(NBUF):
                tb = bb % TBUF
                l = g * NBUF + bb
                wait_gather(bb)

                def _wait_prev_out():
                    pltpu.make_async_copy(
                        tbuf.at[tb, :, :, pl.ds(0, BCH)],
                        out_hbm.at[0, :, pl.ds(0, 8), :], osem[tb]).wait()

                if bb < TBUF:
                    pl.when(g >= 1)(_wait_prev_out)
                else:
                    _wait_prev_out()

                l_splat = jnp.full((LANES,), l, jnp.int32)
                posc = [plsc.load_gather(pos_v, [rows_dq[dq], l_splat])
                        for dq in range(DQ)]

                def per_token(r, cr):
                    cols = jnp.full((LANES,), r, jnp.int32)
                    for dq in range(DQ):
                        v = gbuf[bb, r, pl.ds(dq * LANES, LANES)] + posc[dq]
                        plsc.store_scatter(
                            tbuf.at[tb], [rows_hi[dq], rows_lo[dq], cols], v)
                    return cr

                lax.fori_loop(0, BCH, per_token, 0, unroll=4)

                @pl.when(l + NBUF < MAXLEN)
                def _next_idx():
                    start_idx(l + NBUF, bb)

                nb = (bb + NBUF - 1) % NBUF

                @pl.when(l + NBUF - 1 < MAXLEN)
                def _next_gather():
                    wait_idx(l + NBUF - 1, nb)
                    start_gather(nb)

                pltpu.async_copy(
                    tbuf.at[tb, :, :, pl.ds(0, BCH)],
                    out_hbm.at[l, :, pl.ds(wid * 8, 8), :], osem[tb])
            return carry

        lax.fori_loop(0, NGROUPS, group_body, 0)
        for tb in range(TBUF):
            pltpu.make_async_copy(
                tbuf.at[tb, :, :, pl.ds(0, BCH)],
                out_hbm.at[0, :, pl.ds(0, 8), :], osem[tb]).wait()

    return tok_pos_embed


_kernel_call = _make_kernel()


def kernel(x, token_table, pos_table):
    x3 = (x.astype(jnp.int32)
          .transpose(1, 0)
          .reshape(LH, 8, NW, BCH)
          .transpose(0, 2, 1, 3)
          .reshape(LH, NW * 8, BCH))
    tok_pad = jnp.pad(token_table, ((0, 0), (0, PADW - EMBED_DIM)))
    pos_t = jnp.transpose(pos_table, (1, 0))
    out4 = _kernel_call(x3, tok_pad, pos_t)
    return (out4.reshape(MAXLEN, 8, NW, 8, BCH)
            .transpose(2, 4, 0, 1, 3)
            .reshape(BATCH, MAXLEN, EMBED_DIM))


# speedup vs baseline: 1.4950x; 1.4950x over previous
import functools

import jax
import jax.numpy as jnp
from jax import lax
from jax.experimental import pallas as pl
from jax.experimental.pallas import tpu as pltpu
from jax.experimental.pallas import tpu_sc as plsc

VOCAB = 1000000
MAXLEN = 200
EMBED_DIM = 64
BATCH = 4096

NUM_CORES = 2
NUM_SUBCORES = 16
LANES = 16
NW = NUM_CORES * NUM_SUBCORES
BCH = BATCH // NW
DQ = EMBED_DIM // LANES
PITCH = BCH + 1
NBUF = 5
TBUF = 3
NGROUPS = MAXLEN // NBUF
LH = MAXLEN // 8


def _make_kernel():
    mesh = plsc.VectorSubcoreMesh(core_axis_name="c", subcore_axis_name="s")

    @functools.partial(
        pl.kernel,
        out_type=jax.ShapeDtypeStruct((MAXLEN, 8, NW, 8, BCH), jnp.float32),
        name="tok_pos_embed",
        mesh=mesh,
        scratch_types=[
            pltpu.VMEM((EMBED_DIM, MAXLEN), jnp.float32),
            pltpu.VMEM((LH, 8, BCH), jnp.int32),
            pltpu.VMEM((NBUF, BCH, EMBED_DIM), jnp.float32),
            pltpu.VMEM((TBUF, 8, 8, PITCH), jnp.float32),
            pltpu.SemaphoreType.DMA,
            pltpu.SemaphoreType.DMA,
            pltpu.SemaphoreType.DMA,
            pltpu.SemaphoreType.DMA,
            pltpu.SemaphoreType.DMA,
            pltpu.SemaphoreType.DMA,
            pltpu.SemaphoreType.DMA,
            pltpu.SemaphoreType.DMA,
        ],
        compiler_params=pltpu.CompilerParams(use_tc_tiling_on_sc=False,
                                             needs_layout_passes=False),
    )
    def tok_pos_embed(x_hbm, tok_hbm, pos_hbm, out_hbm,
                      pos_v, idx_v, gbuf, tbuf,
                      g0, g1, g2, g3, g4, o0, o1, o2):
        wid = lax.axis_index("s") * NUM_CORES + lax.axis_index("c")
        gsem = (g0, g1, g2, g3, g4)
        osem = (o0, o1, o2)
        pltpu.sync_copy(pos_hbm, pos_v)
        pltpu.sync_copy(x_hbm.at[:, wid, :, :], idx_v)

        def start_gather(l, bb):
            pltpu.async_copy(tok_hbm.at[idx_v.at[l >> 3, l & 7]],
                             gbuf.at[bb], gsem[bb])

        for bb in range(NBUF):
            start_gather(bb, bb)

        rows_hi = [(lax.iota(jnp.int32, LANES) + dq * LANES) >> 3
                   for dq in range(DQ)]
        rows_lo = [(lax.iota(jnp.int32, LANES) + dq * LANES) & 7
                   for dq in range(DQ)]
        rows_dq = [lax.iota(jnp.int32, LANES) + dq * LANES for dq in range(DQ)]

        def wait_out(tb):
            pltpu.make_async_copy(
                tbuf.at[tb, :, :, pl.ds(0, BCH)],
                out_hbm.at[0, :, 0, :, :], osem[tb]).wait()

        def group_body(g, carry):
            for bb in range(NBUF):
                tb = bb % TBUF
                l = g * NBUF + bb
                pltpu.make_async_copy(
                    tok_hbm.at[idx_v.at[l >> 3, l & 7]],
                    gbuf.at[bb], gsem[bb]).wait()

                if bb < TBUF:
                    pl.when(g >= 1)(lambda: wait_out(tb))
                else:
                    wait_out(tb)

                l_splat = jnp.full((LANES,), l, jnp.int32)
                posc = [plsc.load_gather(pos_v, [rows_dq[dq], l_splat])
                        for dq in range(DQ)]

                def per_token(r, cr):
                    cols = jnp.full((LANES,), r, jnp.int32)
                    for dq in range(DQ):
                        v = gbuf[bb, r, pl.ds(dq * LANES, LANES)] + posc[dq]
                        plsc.store_scatter(
                            tbuf.at[tb], [rows_hi[dq], rows_lo[dq], cols], v)
                    return cr

                lax.fori_loop(0, BCH, per_token, 0, unroll=4)

                @pl.when(g < NGROUPS - 1)
                def _next_gather():
                    start_gather(l + NBUF, bb)

                pltpu.async_copy(
                    tbuf.at[tb, :, :, pl.ds(0, BCH)],
                    out_hbm.at[l, :, wid, :, :], osem[tb])
            return carry

        lax.fori_loop(0, NGROUPS, group_body, 0)
        for tb in range(TBUF):
            wait_out(tb)

    return tok_pos_embed


_kernel_call = _make_kernel()


def kernel(x, token_table, pos_table):
    x4 = (x.astype(jnp.int32)
          .transpose(1, 0)
          .reshape(LH, 8, NW, BCH)
          .transpose(0, 2, 1, 3))
    pos_t = jnp.transpose(pos_table, (1, 0))
    out5 = _kernel_call(x4, token_table, pos_t)
    return (out5.transpose(2, 4, 0, 1, 3)
            .reshape(BATCH, MAXLEN, EMBED_DIM))
